# baseline (speedup 1.0000x reference)
import jax
import jax.numpy as jnp
from jax import lax
from jax.experimental import pallas as pl
from jax.experimental.pallas import tpu as pltpu
from jax.experimental.pallas import tpu_sc as plsc

_N_EMBED = 64
_SCALE = float(_N_EMBED) ** 0.5

_CHUNK = 128
_B = 16384 * 50
_NW = 32
_ROWS_PER_W = _B // _NW
_CHUNKS_PER_W = _ROWS_PER_W // _CHUNK
_NBUF = 4
_LA = 2


def _gather_body(x_hbm, table_hbm, out_hbm, idx_v, rows_v, gsems, ssems):
    wid = lax.axis_index("s") * 2 + lax.axis_index("c")

    pltpu.sync_copy(x_hbm.at[pl.ds(wid * _CHUNKS_PER_W, _CHUNKS_PER_W)], idx_v)

    row_base = wid * _ROWS_PER_W

    def start_gather(g, b):
        pltpu.make_async_copy(
            table_hbm.at[idx_v.at[g]], rows_v.at[b], gsems.at[b]
        ).start()

    def wait_gather(b):
        pltpu.make_async_copy(
            table_hbm.at[idx_v.at[0]], rows_v.at[b], gsems.at[b]
        ).wait()

    def start_store(g, b):
        pltpu.make_async_copy(
            rows_v.at[b], out_hbm.at[pl.ds(row_base + g * _CHUNK, _CHUNK)],
            ssems.at[b],
        ).start()

    def wait_store(b):
        pltpu.make_async_copy(
            rows_v.at[b], out_hbm.at[pl.ds(row_base, _CHUNK)], ssems.at[b]
        ).wait()


    def emit(j, b, g, need_store_wait):
        wait_gather(b)
        start_store(j, b)
        if g is not None:
            b2 = (b + _LA) % _NBUF
            if need_store_wait:
                wait_store(b2)
            start_gather(g, b2)

    for g in range(_LA):
        start_gather(g, g % _NBUF)

    for j in range(_NBUF):
        emit(j, j % _NBUF, j + _LA, j + _LA >= _NBUF)

    n_groups = (_CHUNKS_PER_W - _NBUF - _LA) // _NBUF

    def steady(t, _):
        j0 = _NBUF + t * _NBUF
        for i in range(_NBUF):
            emit(j0 + i, i, j0 + i + _LA, True)
        return 0

    lax.fori_loop(0, n_groups, steady, 0)

    for j in range(_NBUF + n_groups * _NBUF, _CHUNKS_PER_W):
        g = j + _LA
        emit(j, j % _NBUF, g if g < _CHUNKS_PER_W else None, True)

    for b in range(_NBUF):
        wait_store(b)


@jax.jit
def _embed(x_flat, embed_weight):
    mesh = plsc.VectorSubcoreMesh(core_axis_name="c", subcore_axis_name="s")
    run = pl.kernel(
        _gather_body,
        out_type=jax.ShapeDtypeStruct((_B, _N_EMBED), jnp.float32),
        mesh=mesh,
        scratch_types=[
            pltpu.VMEM((_CHUNKS_PER_W, _CHUNK), jnp.int32),
            pltpu.VMEM((_NBUF, _CHUNK, _N_EMBED), jnp.float32),
            pltpu.SemaphoreType.DMA((_NBUF,)),
            pltpu.SemaphoreType.DMA((_NBUF,)),
        ],
        compiler_params=pltpu.CompilerParams(use_tc_tiling_on_sc=False),
    )
    return run(x_flat, embed_weight)


def kernel(x, embed_weight):
    x_flat = x.reshape(_B // _CHUNK, _CHUNK).astype(jnp.int32)
    rows = _embed(x_flat, embed_weight)
    return rows.reshape(x.shape[0], x.shape[1], _N_EMBED) * _SCALE


# speedup vs baseline: 1.5117x; 1.5117x over previous
import jax
import jax.numpy as jnp
from jax import lax
from jax.experimental import pallas as pl
from jax.experimental.pallas import tpu as pltpu
from jax.experimental.pallas import tpu_sc as plsc

_D = 64
_DP = 128
_SP = 56
_SCALE = float(_D) ** 0.5
_L = 16

_NW = 32
_SEQ = 16384
_SLEN = 50
_SEQ_PER_W = _SEQ // _NW
_CSEQ = 4
_CIDX = _CSEQ * _SLEN
_CHUNKS_PER_W = _SEQ_PER_W // _CSEQ
_IDX_PER_W = _SEQ_PER_W * _SLEN
_NBUF = 2
_G_SPLITS = ((0, 96), (96, 104))


def _gather_body(x_hbm, table_hbm, out_hbm, idx_v, grow_v, stage_v, gsems, ssems):
    wid = lax.axis_index("s") * 2 + lax.axis_index("c")

    pltpu.sync_copy(x_hbm.at[pl.ds(wid * _IDX_PER_W, _IDX_PER_W)], idx_v)

    zrow_base = wid * _SEQ_PER_W * _SP

    def start_gather(g, b):
        for off, n in _G_SPLITS:
            pltpu.make_async_copy(
                table_hbm.at[idx_v.at[pl.ds(g * _CIDX + off, n)]],
                grow_v.at[b, pl.ds(off, n)],
                gsems.at[b],
            ).start()

    def wait_gather(b):
        for off, n in _G_SPLITS:
            pltpu.make_async_copy(
                table_hbm.at[idx_v.at[pl.ds(off, n)]],
                grow_v.at[b, pl.ds(off, n)],
                gsems.at[b],
            ).wait()

    def start_stores(g, b):
        for k in range(_CSEQ):
            pltpu.make_async_copy(
                stage_v.at[b, k],
                out_hbm.at[pl.ds(zrow_base + (g * _CSEQ + k) * _SP, _SLEN)],
                ssems.at[b],
            ).start()

    def wait_stores(b):
        for k in range(_CSEQ):
            pltpu.make_async_copy(
                stage_v.at[b, k],
                out_hbm.at[pl.ds(zrow_base, _SLEN)],
                ssems.at[b],
            ).wait()

    def expand(b):
        for k in range(_CSEQ):
            def body(j, _):
                for c in range(_D // _L):
                    sl = pl.ds(c * _L, _L)
                    stage_v[b, k, j, sl] = grow_v[b, 50 * k + j, sl] * _SCALE
                return 0

            lax.fori_loop(0, _SLEN, body, 0)

    def step(g, b, has_next, need_store_wait):
        wait_gather(b)
        if has_next:
            start_gather(g + 1, 1 - b)
        if need_store_wait:
            wait_stores(b)
        expand(b)
        start_stores(g, b)

    start_gather(0, 0)
    step(0, 0, True, False)
    step(1, 1, True, False)

    def steady(t, _):
        g0 = 2 + t * _NBUF
        step(g0, 0, True, True)
        step(g0 + 1, 1, True, True)
        return 0

    n_groups = (_CHUNKS_PER_W - 2 - 2) // _NBUF
    lax.fori_loop(0, n_groups, steady, 0)

    step(_CHUNKS_PER_W - 2, 0, True, True)
    step(_CHUNKS_PER_W - 1, 1, False, True)

    for b in range(_NBUF):
        wait_stores(b)


@jax.jit
def _embed(x_lin, embed_weight):
    mesh = plsc.VectorSubcoreMesh(core_axis_name="c", subcore_axis_name="s")
    run = pl.kernel(
        _gather_body,
        out_type=jax.ShapeDtypeStruct((_SEQ * _SP, _DP), jnp.float32),
        mesh=mesh,
        scratch_types=[
            pltpu.VMEM((_IDX_PER_W,), jnp.int32),
            pltpu.VMEM((_NBUF, _CIDX, _D), jnp.float32),
            pltpu.VMEM((_NBUF, _CSEQ, _SLEN, _DP), jnp.float32),
            pltpu.SemaphoreType.DMA((_NBUF,)),
            pltpu.SemaphoreType.DMA((_NBUF,)),
        ],
        compiler_params=pltpu.CompilerParams(use_tc_tiling_on_sc=False),
    )
    return run(x_lin, embed_weight)


def kernel(x, embed_weight):
    x_lin = x.reshape(_SEQ * _SLEN).astype(jnp.int32)
    z = _embed(x_lin, embed_weight)
    return z.reshape(_SEQ, _SP, _DP)[:, :_SLEN, :_D]
